# Initial kernel scaffold; baseline (speedup 1.0000x reference)
import jax
import jax.numpy as jnp
from jax.experimental import pallas as pl


def kernel(x, edge_index, adj_values, W, b):
    raise NotImplementedError("write your pallas kernel here")


# speedup vs baseline: 1.0944x; 1.0944x over previous
import functools

import jax
import jax.numpy as jnp
from jax.experimental import pallas as pl
from jax.experimental.pallas import tpu as pltpu

N = 10000
E = 320000
D = 128
C = 1.0
MIN_NORM = 1e-15
EPS_F32 = 4e-3


def _rownorm(v):
    return jnp.maximum(jnp.sqrt(jnp.sum(v * v, axis=-1, keepdims=True)), MIN_NORM)


def _proj(v):
    nrm = _rownorm(v)
    maxnorm = 1.0 - EPS_F32
    return jnp.where(nrm > maxnorm, v / nrm * maxnorm, v)


def _support_body(x_ref, w_ref, y0_ref, o_ref):
    x = x_ref[...]
    pn = _rownorm(x)
    z = jnp.clip(pn, -1.0 + 1e-7, 1.0 - 1e-7)
    art = 0.5 * jnp.log((1.0 + z) / (1.0 - z))
    u = (art / pn) * x
    s = jnp.dot(u, w_ref[...], preferred_element_type=jnp.float32)
    y = y0_ref[...]
    x2 = jnp.sum(s * s, axis=-1, keepdims=True)
    y2 = jnp.sum(y * y, axis=-1, keepdims=True)
    xy = jnp.sum(s * y, axis=-1, keepdims=True)
    num = (1.0 + 2.0 * xy + y2) * s + (1.0 - x2) * y
    den = jnp.maximum(1.0 + 2.0 * xy + x2 * y2, MIN_NORM)
    o_ref[...] = _proj(num / den)


def _final_body(p_ref, o_ref):
    u = p_ref[...]
    un = _rownorm(u)
    e = jnp.tanh(un) * u / un
    o_ref[...] = _proj(e)


def _compute_support(x, W, y0):
    return pl.pallas_call(
        _support_body,
        out_shape=jax.ShapeDtypeStruct((N, D), jnp.float32),
    )(x, W, y0)


def _finalize(agg):
    return pl.pallas_call(
        _final_body,
        out_shape=jax.ShapeDtypeStruct((N, D), jnp.float32),
    )(agg)


def kernel(x, edge_index, adj_values, W, b):
    bias = b.reshape(1, -1)
    bn = jnp.maximum(jnp.sqrt(jnp.sum(bias * bias, axis=-1, keepdims=True)), MIN_NORM)
    hyp_bias = jnp.tanh(bn) * bias / bn
    hb_n = jnp.maximum(jnp.sqrt(jnp.sum(hyp_bias * hyp_bias, axis=-1, keepdims=True)), MIN_NORM)
    hyp_bias = jnp.where(hb_n > 1.0 - EPS_F32, hyp_bias / hb_n * (1.0 - EPS_F32), hyp_bias)

    support = _compute_support(x, W, hyp_bias)

    row = edge_index[0].astype(jnp.int32)
    col = edge_index[1].astype(jnp.int32)
    gathered = adj_values[:, None] * jnp.take(support, col, axis=0)
    agg = jax.ops.segment_sum(gathered, row, num_segments=N)

    return _finalize(agg)


# speedup vs baseline: 3.0565x; 2.7929x over previous
import functools

import jax
import jax.numpy as jnp
from jax import lax
from jax.experimental import pallas as pl
from jax.experimental.pallas import tpu as pltpu
from jax.experimental.pallas import tpu_sc as plsc

N = 10000
E = 320000
D = 128
MIN_NORM = 1e-15
EPS_F32 = 4e-3

_NC = 2
_NS = 16
_TILES = _NC * _NS
_PER_TILE = E // _TILES
_N_PAD = 10240
_ROWS_PER_SUB = _N_PAD // _NS
_SC_K = 80
_SC_CHUNKS = _PER_TILE // _SC_K


def _rownorm(v):
    return jnp.maximum(jnp.sqrt(jnp.sum(v * v, axis=-1, keepdims=True)), MIN_NORM)


def _proj(v):
    nrm = _rownorm(v)
    maxnorm = 1.0 - EPS_F32
    return jnp.where(nrm > maxnorm, v / nrm * maxnorm, v)


def _support_body(x_ref, w_ref, y0_ref, o_ref):
    x = x_ref[...]
    pn = _rownorm(x)
    z = jnp.clip(pn, -1.0 + 1e-7, 1.0 - 1e-7)
    art = 0.5 * jnp.log((1.0 + z) / (1.0 - z))
    u = (art / pn) * x
    s = jnp.dot(u, w_ref[...], preferred_element_type=jnp.float32)
    y = y0_ref[...]
    x2 = jnp.sum(s * s, axis=-1, keepdims=True)
    y2 = jnp.sum(y * y, axis=-1, keepdims=True)
    xy = jnp.sum(s * y, axis=-1, keepdims=True)
    num = (1.0 + 2.0 * xy + y2) * s + (1.0 - x2) * y
    den = jnp.maximum(1.0 + 2.0 * xy + x2 * y2, MIN_NORM)
    o_ref[...] = _proj(num / den)


def _final_body(p_ref, o_ref):
    u = p_ref[0, :N, :] + p_ref[1, :N, :]
    un = _rownorm(u)
    e = jnp.tanh(un) * u / un
    o_ref[...] = _proj(e)


def _compute_support(x, W, y0):
    return pl.pallas_call(
        _support_body,
        out_shape=jax.ShapeDtypeStruct((N, D), jnp.float32),
    )(x, W, y0)


def _finalize(partials):
    return pl.pallas_call(
        _final_body,
        out_shape=jax.ShapeDtypeStruct((N, D), jnp.float32),
    )(partials)


def _sc_aggregate(support, col, row, vals16, zeros):
    mesh = plsc.VectorSubcoreMesh(core_axis_name="c", subcore_axis_name="s")

    @functools.partial(
        pl.kernel,
        out_type=jax.ShapeDtypeStruct((_NC, _N_PAD, D), jnp.float32),
        mesh=mesh,
        scratch_types=[
            pltpu.VMEM((_SC_K,), jnp.int32),
            pltpu.VMEM((_SC_K,), jnp.int32),
            pltpu.VMEM((_SC_K, 16), jnp.float32),
            pltpu.VMEM((_SC_K, D), jnp.float32),
            pltpu.VMEM_SHARED((_N_PAD, D), jnp.float32),
            pltpu.SemaphoreType.DMA,
        ],
    )
    def sc_kernel(sup_hbm, col_hbm, row_hbm, vals_hbm, z_hbm, out_hbm,
                  colv, rowv, valsv, rows, acc, sem):
        cid = lax.axis_index("c")
        sid = lax.axis_index("s")
        zbase = sid * _ROWS_PER_SUB
        pltpu.sync_copy(z_hbm.at[pl.ds(zbase, _ROWS_PER_SUB)],
                        acc.at[pl.ds(zbase, _ROWS_PER_SUB)])
        plsc.subcore_barrier()

        base = (cid * _NS + sid) * _PER_TILE

        @pl.loop(0, _SC_CHUNKS)
        def _chunk(i):
            off = base + i * _SC_K
            pltpu.sync_copy(col_hbm.at[pl.ds(off, _SC_K)], colv)
            pltpu.sync_copy(row_hbm.at[pl.ds(off, _SC_K)], rowv)
            pltpu.sync_copy(vals_hbm.at[pl.ds(off, _SC_K)], valsv)
            pltpu.async_copy(sup_hbm.at[colv], rows, sem).wait()

            @pl.loop(0, _SC_K)
            def _scale(j):
                v = valsv[j]
                for t in range(D // 16):
                    rows[j, pl.ds(t * 16, 16)] = rows[j, pl.ds(t * 16, 16)] * v

            pltpu.sync_copy(rows, acc.at[rowv], add=True)

        plsc.subcore_barrier()
        pltpu.sync_copy(acc.at[pl.ds(zbase, _ROWS_PER_SUB)],
                        out_hbm.at[cid, pl.ds(zbase, _ROWS_PER_SUB)])

    return sc_kernel(support, col, row, vals16, zeros)


def kernel(x, edge_index, adj_values, W, b):
    bias = b.reshape(1, -1).astype(jnp.float32)
    bn = jnp.maximum(jnp.sqrt(jnp.sum(bias * bias, axis=-1, keepdims=True)), MIN_NORM)
    hyp_bias = jnp.tanh(bn) * bias / bn
    hb_n = jnp.maximum(jnp.sqrt(jnp.sum(hyp_bias * hyp_bias, axis=-1, keepdims=True)), MIN_NORM)
    hyp_bias = jnp.where(hb_n > 1.0 - EPS_F32, hyp_bias / hb_n * (1.0 - EPS_F32), hyp_bias)

    support = _compute_support(x, W, hyp_bias)

    row = edge_index[0].astype(jnp.int32)
    col = edge_index[1].astype(jnp.int32)
    vals16 = jnp.broadcast_to(adj_values.astype(jnp.float32)[:, None], (E, 16))
    zeros = jnp.zeros((_N_PAD, D), jnp.float32)

    partials = _sc_aggregate(support, col, row, vals16, zeros)

    return _finalize(partials)


# speedup vs baseline: 4.2989x; 1.4065x over previous
import functools

import jax
import jax.numpy as jnp
from jax import lax
from jax.experimental import pallas as pl
from jax.experimental.pallas import tpu as pltpu
from jax.experimental.pallas import tpu_sc as plsc

N = 10000
E = 320000
D = 128
MIN_NORM = 1e-15
EPS_F32 = 4e-3

_NC = 2
_NS = 16
_TILES = _NC * _NS
_PER_TILE = E // _TILES
_N_PAD = 10240
_ROWS_PER_SUB = _N_PAD // _NS
_SC_K = 40
_NBUF = 4
_NCH = 252
_PER_TILE_PAD = _NCH * _SC_K


def _rownorm(v):
    return jnp.maximum(jnp.sqrt(jnp.sum(v * v, axis=-1, keepdims=True)), MIN_NORM)


def _proj(v):
    nrm = _rownorm(v)
    maxnorm = 1.0 - EPS_F32
    return jnp.where(nrm > maxnorm, v / nrm * maxnorm, v)


def _support_body(x_ref, w_ref, y0_ref, o_ref):
    x = x_ref[...]
    pn = _rownorm(x)
    z = jnp.clip(pn, -1.0 + 1e-7, 1.0 - 1e-7)
    art = 0.5 * jnp.log((1.0 + z) / (1.0 - z))
    u = (art / pn) * x
    s = jnp.dot(u, w_ref[...], preferred_element_type=jnp.float32)
    y = y0_ref[...]
    x2 = jnp.sum(s * s, axis=-1, keepdims=True)
    y2 = jnp.sum(y * y, axis=-1, keepdims=True)
    xy = jnp.sum(s * y, axis=-1, keepdims=True)
    num = (1.0 + 2.0 * xy + y2) * s + (1.0 - x2) * y
    den = jnp.maximum(1.0 + 2.0 * xy + x2 * y2, MIN_NORM)
    o_ref[...] = _proj(num / den)


def _final_body(p_ref, o_ref):
    u = p_ref[0, :N, :] + p_ref[1, :N, :]
    un = _rownorm(u)
    e = jnp.tanh(un) * u / un
    o_ref[...] = _proj(e)


def _compute_support(x, W, y0):
    return pl.pallas_call(
        _support_body,
        out_shape=jax.ShapeDtypeStruct((N, D), jnp.float32),
    )(x, W, y0)


def _finalize(partials):
    return pl.pallas_call(
        _final_body,
        out_shape=jax.ShapeDtypeStruct((N, D), jnp.float32),
    )(partials)


def _sc_aggregate(support, col3, row3, vals16, zeros):
    mesh = plsc.VectorSubcoreMesh(core_axis_name="c", subcore_axis_name="s")

    @functools.partial(
        pl.kernel,
        out_type=jax.ShapeDtypeStruct((_NC, _N_PAD, D), jnp.float32),
        mesh=mesh,
        scratch_types=[
            [pltpu.VMEM((_SC_K,), jnp.int32) for _ in range(_NBUF)],
            [pltpu.VMEM((_SC_K,), jnp.int32) for _ in range(_NBUF)],
            [pltpu.VMEM((_SC_K, 16), jnp.float32) for _ in range(_NBUF)],
            [pltpu.VMEM((_SC_K, D), jnp.float32) for _ in range(_NBUF)],
            pltpu.VMEM_SHARED((_N_PAD, D), jnp.float32),
            [pltpu.SemaphoreType.DMA for _ in range(_NBUF)],
            [pltpu.SemaphoreType.DMA for _ in range(_NBUF)],
            [pltpu.SemaphoreType.DMA for _ in range(_NBUF)],
        ],
    )
    def sc_kernel(sup_hbm, col_hbm, row_hbm, vals_hbm, z_hbm, out_hbm,
                  colb, rowb, valsv, rows, acc, isem, gsem, ssem):
        cid = lax.axis_index("c")
        sid = lax.axis_index("s")
        wid = cid * _NS + sid
        zbase = sid * _ROWS_PER_SUB
        pltpu.sync_copy(z_hbm.at[pl.ds(zbase, _ROWS_PER_SUB)],
                        acc.at[pl.ds(zbase, _ROWS_PER_SUB)])
        plsc.subcore_barrier()

        def start_idx(i, b):
            pltpu.async_copy(col_hbm.at[wid, i], colb[b], isem[b])
            pltpu.async_copy(row_hbm.at[wid, i], rowb[b], isem[b])
            pltpu.async_copy(vals_hbm.at[wid, pl.ds(i * _SC_K, _SC_K)],
                             valsv[b], isem[b])

        def wait_idx(b):
            pltpu.make_async_copy(col_hbm.at[wid, 0], colb[b], isem[b]).wait()
            pltpu.make_async_copy(row_hbm.at[wid, 0], rowb[b], isem[b]).wait()
            pltpu.make_async_copy(vals_hbm.at[wid, pl.ds(0, _SC_K)],
                                  valsv[b], isem[b]).wait()

        def start_gather(b):
            pltpu.async_copy(sup_hbm.at[colb[b]], rows[b], gsem[b])

        def wait_gather(b):
            pltpu.make_async_copy(sup_hbm.at[colb[b]], rows[b], gsem[b]).wait()

        def wait_scatter(b):
            pltpu.make_async_copy(rows[b], acc.at[rowb[b]], ssem[b]).wait()

        start_idx(0, 0)
        start_idx(1, 1)
        wait_idx(0)
        start_gather(0)
        start_idx(2, 2)
        start_idx(3, 3)
        wait_idx(1)
        start_gather(1)

        @pl.loop(0, _NCH // _NBUF)
        def _grp(p):
            for b in range(_NBUF):
                i = p * _NBUF + b
                wait_gather(b)

                @plsc.parallel_loop(0, _SC_K, unroll=4)
                def _scale(j):
                    v = valsv[b][j]
                    for t in range(D // 16):
                        rows[b][j, pl.ds(t * 16, 16)] = (
                            rows[b][j, pl.ds(t * 16, 16)] * v)

                pltpu.async_copy(rows[b], acc.at[rowb[b]], ssem[b], add=True)

                bp = (b + _NBUF - 1) % _NBUF

                @pl.when(i >= 1)
                def _recycle():
                    wait_scatter(bp)

                    @pl.when(i + _NBUF - 1 < _NCH)
                    def _():
                        start_idx(i + _NBUF - 1, bp)

                bg = (b + 2) % _NBUF

                @pl.when(i + 2 < _NCH)
                def _gather_next():
                    wait_idx(bg)
                    start_gather(bg)

        wait_scatter((_NCH - 1) % _NBUF)

        plsc.subcore_barrier()
        pltpu.sync_copy(acc.at[pl.ds(zbase, _ROWS_PER_SUB)],
                        out_hbm.at[cid, pl.ds(zbase, _ROWS_PER_SUB)])

    return sc_kernel(support, col3, row3, vals16, zeros)


def _pad_tiles(a, fill):
    a = a.reshape(_TILES, _PER_TILE)
    return jnp.pad(a, ((0, 0), (0, _PER_TILE_PAD - _PER_TILE)),
                   constant_values=fill)


def kernel(x, edge_index, adj_values, W, b):
    bias = b.reshape(1, -1).astype(jnp.float32)
    bn = jnp.maximum(jnp.sqrt(jnp.sum(bias * bias, axis=-1, keepdims=True)), MIN_NORM)
    hyp_bias = jnp.tanh(bn) * bias / bn
    hb_n = jnp.maximum(jnp.sqrt(jnp.sum(hyp_bias * hyp_bias, axis=-1, keepdims=True)), MIN_NORM)
    hyp_bias = jnp.where(hb_n > 1.0 - EPS_F32, hyp_bias / hb_n * (1.0 - EPS_F32), hyp_bias)

    support = _compute_support(x, W, hyp_bias)

    row = edge_index[0].astype(jnp.int32)
    col = edge_index[1].astype(jnp.int32)
    vals = adj_values.astype(jnp.float32)

    col3 = _pad_tiles(col, 0).reshape(_TILES, _NCH, _SC_K)
    row3 = _pad_tiles(row, _N_PAD - 1).reshape(_TILES, _NCH, _SC_K)
    vals_p = _pad_tiles(vals, 0.0)
    vals16 = jnp.broadcast_to(vals_p[..., None], (_TILES, _PER_TILE_PAD, 16))
    zeros = jnp.zeros((_N_PAD, D), jnp.float32)

    partials = _sc_aggregate(support, col3, row3, vals16, zeros)

    return _finalize(partials)


# speedup vs baseline: 7.7143x; 1.7945x over previous
import functools

import jax
import jax.numpy as jnp
from jax import lax
from jax.experimental import pallas as pl
from jax.experimental.pallas import tpu as pltpu
from jax.experimental.pallas import tpu_sc as plsc

N = 10000
E = 320000
D = 128
MIN_NORM = 1e-15
EPS_F32 = 4e-3

_NC = 2
_NS = 16
_TILES = _NC * _NS
_PER_TILE = E // _TILES
_N_PAD = 10240
_ROWS_PER_SUB = _N_PAD // _NS
_SC_K = 80
_NBUF = 4
_NCH = _PER_TILE // _SC_K


def _rownorm(v):
    return jnp.maximum(jnp.sqrt(jnp.sum(v * v, axis=-1, keepdims=True)), MIN_NORM)


def _proj(v):
    nrm = _rownorm(v)
    maxnorm = 1.0 - EPS_F32
    return jnp.where(nrm > maxnorm, v / nrm * maxnorm, v)


def _support_body(x_ref, w_ref, b_ref, o_ref):
    x = x_ref[...]
    pn = _rownorm(x)
    z = jnp.clip(pn, -1.0 + 1e-7, 1.0 - 1e-7)
    art = 0.5 * jnp.log((1.0 + z) / (1.0 - z))
    u = (art / pn) * x
    s = jnp.dot(u, w_ref[...], preferred_element_type=jnp.float32)
    bias = b_ref[...]
    y = _proj(jnp.tanh(_rownorm(bias)) * bias / _rownorm(bias))
    x2 = jnp.sum(s * s, axis=-1, keepdims=True)
    y2 = jnp.sum(y * y, axis=-1, keepdims=True)
    xy = jnp.sum(s * y, axis=-1, keepdims=True)
    num = (1.0 + 2.0 * xy + y2) * s + (1.0 - x2) * y
    den = jnp.maximum(1.0 + 2.0 * xy + x2 * y2, MIN_NORM)
    o_ref[...] = _proj(num / den)


def _final_body(p_ref, o_ref):
    u = p_ref[0, :N, :] + p_ref[1, :N, :]
    un = _rownorm(u)
    e = jnp.tanh(un) * u / un
    o_ref[...] = _proj(e)


def _compute_support(x, W, b2d):
    return pl.pallas_call(
        _support_body,
        out_shape=jax.ShapeDtypeStruct((N, D), jnp.float32),
    )(x, W, b2d)


def _finalize(partials):
    return pl.pallas_call(
        _final_body,
        out_shape=jax.ShapeDtypeStruct((N, D), jnp.float32),
    )(partials)


def _sc_aggregate(support, col3, row3, vals16, zeros):
    mesh = plsc.VectorSubcoreMesh(core_axis_name="c", subcore_axis_name="s")

    @functools.partial(
        pl.kernel,
        out_type=jax.ShapeDtypeStruct((_NC, _N_PAD, D), jnp.float32),
        mesh=mesh,
        scratch_types=[
            [pltpu.VMEM((_SC_K,), jnp.int32) for _ in range(_NBUF)],
            [pltpu.VMEM((_SC_K,), jnp.int32) for _ in range(_NBUF)],
            [pltpu.VMEM((_SC_K * 16,), jnp.float32) for _ in range(_NBUF)],
            [pltpu.VMEM((_SC_K, D), jnp.float32) for _ in range(_NBUF)],
            pltpu.VMEM_SHARED((_N_PAD, D), jnp.float32),
            [pltpu.SemaphoreType.DMA for _ in range(_NBUF)],
            [pltpu.SemaphoreType.DMA for _ in range(_NBUF)],
            [pltpu.SemaphoreType.DMA for _ in range(_NBUF)],
        ],
    )
    def sc_kernel(sup_hbm, col_hbm, row_hbm, vals_hbm, z_hbm, out_hbm,
                  colb, rowb, valsv, rows, acc, isem, gsem, ssem):
        cid = lax.axis_index("c")
        sid = lax.axis_index("s")
        wid = cid * _NS + sid
        zbase = sid * _ROWS_PER_SUB
        pltpu.sync_copy(z_hbm.at[pl.ds(zbase, _ROWS_PER_SUB)],
                        acc.at[pl.ds(zbase, _ROWS_PER_SUB)])
        plsc.subcore_barrier()

        def start_idx(i, b):
            pltpu.async_copy(col_hbm.at[wid, i], colb[b], isem[b])
            pltpu.async_copy(row_hbm.at[wid, i], rowb[b], isem[b])
            pltpu.async_copy(vals_hbm.at[wid, pl.ds(i * _SC_K * 16, _SC_K * 16)],
                             valsv[b], isem[b])

        def wait_idx(b):
            pltpu.make_async_copy(col_hbm.at[wid, 0], colb[b], isem[b]).wait()
            pltpu.make_async_copy(row_hbm.at[wid, 0], rowb[b], isem[b]).wait()
            pltpu.make_async_copy(vals_hbm.at[wid, pl.ds(0, _SC_K * 16)],
                                  valsv[b], isem[b]).wait()

        def start_gather(b):
            pltpu.async_copy(sup_hbm.at[colb[b]], rows[b], gsem[b])

        def wait_gather(b):
            pltpu.make_async_copy(sup_hbm.at[colb[b]], rows[b], gsem[b]).wait()

        def start_scatter(b):
            pltpu.async_copy(rows[b], acc.at[rowb[b]], ssem[b], add=True)

        def wait_scatter(b):
            pltpu.make_async_copy(rows[b], acc.at[rowb[b]], ssem[b]).wait()

        def scale(b):
            @plsc.parallel_loop(0, _SC_K, unroll=4)
            def _scale(j):
                v = valsv[b][pl.ds(j * 16, 16)]
                for t in range(D // 16):
                    rows[b][j, pl.ds(t * 16, 16)] = (
                        rows[b][j, pl.ds(t * 16, 16)] * v)

        start_idx(0, 0)
        start_idx(1, 1)
        wait_idx(0)
        start_gather(0)
        start_idx(2, 2)
        start_idx(3, 3)
        wait_idx(1)
        start_gather(1)

        @pl.loop(0, (_NCH - 1) // _NBUF)
        def _grp(p):
            for b in range(_NBUF):
                i = p * _NBUF + b
                wait_gather(b)
                scale(b)
                start_scatter(b)

                bp = (b + _NBUF - 1) % _NBUF

                @pl.when(i >= 1)
                def _recycle():
                    wait_scatter(bp)

                    @pl.when(i + _NBUF - 1 < _NCH)
                    def _():
                        start_idx(i + _NBUF - 1, bp)

                bg = (b + 2) % _NBUF

                @pl.when(i + 2 < _NCH)
                def _gather_next():
                    wait_idx(bg)
                    start_gather(bg)

        tb = (_NCH - 1) % _NBUF
        wait_gather(tb)
        scale(tb)
        start_scatter(tb)
        wait_scatter((tb + _NBUF - 1) % _NBUF)
        wait_scatter(tb)

        plsc.subcore_barrier()
        pltpu.sync_copy(acc.at[pl.ds(zbase, _ROWS_PER_SUB)],
                        out_hbm.at[cid, pl.ds(zbase, _ROWS_PER_SUB)])

    return sc_kernel(support, col3, row3, vals16, zeros)


def kernel(x, edge_index, adj_values, W, b):
    support = _compute_support(x, W, b.reshape(1, -1).astype(jnp.float32))

    row = edge_index[0].astype(jnp.int32)
    col = edge_index[1].astype(jnp.int32)
    vals = adj_values.astype(jnp.float32)

    col3 = col.reshape(_TILES, _NCH, _SC_K)
    row3 = row.reshape(_TILES, _NCH, _SC_K)
    vals16 = jnp.broadcast_to(vals[:, None], (E, 16)).reshape(_TILES, _PER_TILE * 16)
    zeros = jnp.zeros((_N_PAD, D), jnp.float32)

    partials = _sc_aggregate(support, col3, row3, vals16, zeros)

    return _finalize(partials)


# speedup vs baseline: 10.6647x; 1.3825x over previous
import dataclasses
import functools

import jax
import jax.numpy as jnp
from jax import lax
from jax.experimental import pallas as pl
from jax.experimental.pallas import tpu as pltpu
from jax.experimental.pallas import tpu_sc as plsc

N = 10000
E = 320000
D = 128
MIN_NORM = 1e-15
EPS_F32 = 4e-3

_NC = 2
_NS = 16
_TILES = _NC * _NS
_PER_TILE = E // _TILES
_N_PAD = 10240
_ROWS_PER_SUB = _N_PAD // _NS
_SC_K = 80
_NBUF = 4
_NCH = _PER_TILE // _SC_K


def _rownorm(v):
    return jnp.maximum(jnp.sqrt(jnp.sum(v * v, axis=-1, keepdims=True)), MIN_NORM)


def _proj(v):
    nrm = _rownorm(v)
    maxnorm = 1.0 - EPS_F32
    return jnp.where(nrm > maxnorm, v / nrm * maxnorm, v)


def _support_body(x_ref, w_ref, b_ref, o_ref):
    x = x_ref[...]
    pn = _rownorm(x)
    z = jnp.clip(pn, -1.0 + 1e-7, 1.0 - 1e-7)
    art = 0.5 * jnp.log((1.0 + z) / (1.0 - z))
    u = (art / pn) * x
    s = jnp.dot(u, w_ref[...], preferred_element_type=jnp.float32)
    bias = b_ref[...]
    y = _proj(jnp.tanh(_rownorm(bias)) * bias / _rownorm(bias))
    x2 = jnp.sum(s * s, axis=-1, keepdims=True)
    y2 = jnp.sum(y * y, axis=-1, keepdims=True)
    xy = jnp.sum(s * y, axis=-1, keepdims=True)
    num = (1.0 + 2.0 * xy + y2) * s + (1.0 - x2) * y
    den = jnp.maximum(1.0 + 2.0 * xy + x2 * y2, MIN_NORM)
    o_ref[...] = _proj(num / den)


def _final_body(p_ref, o_ref):
    u = p_ref[0, :N, :] + p_ref[1, :N, :]
    un = _rownorm(u)
    e = jnp.tanh(un) * u / un
    o_ref[...] = _proj(e)


def _compute_support(x, W, b2d):
    return pl.pallas_call(
        _support_body,
        out_shape=jax.ShapeDtypeStruct((N, D), jnp.float32),
    )(x, W, b2d)


def _finalize(partials):
    return pl.pallas_call(
        _final_body,
        out_shape=jax.ShapeDtypeStruct((N, D), jnp.float32),
    )(partials)


def _sc_aggregate(support, col3, row3, vals16, zeros):
    mesh = plsc.VectorSubcoreMesh(core_axis_name="c", subcore_axis_name="s")
    cp = pltpu.CompilerParams()
    if "needs_layout_passes" in pltpu.CompilerParams.__dataclass_fields__:
        cp = dataclasses.replace(cp, needs_layout_passes=False)

    @functools.partial(
        pl.kernel,
        out_type=jax.ShapeDtypeStruct((_NC, _N_PAD, D), jnp.float32),
        mesh=mesh,
        compiler_params=cp,
        scratch_types=[
            [pltpu.VMEM((_SC_K,), jnp.int32) for _ in range(_NBUF)],
            [pltpu.VMEM((_SC_K,), jnp.int32) for _ in range(_NBUF)],
            [pltpu.VMEM((_SC_K,), jnp.float32) for _ in range(_NBUF)],
            [pltpu.VMEM((_SC_K, D), jnp.float32) for _ in range(_NBUF)],
            pltpu.VMEM_SHARED((_N_PAD, D), jnp.float32),
            [pltpu.SemaphoreType.DMA for _ in range(_NBUF)],
            [pltpu.SemaphoreType.DMA for _ in range(_NBUF)],
            [pltpu.SemaphoreType.DMA for _ in range(_NBUF)],
        ],
    )
    def sc_kernel(sup_hbm, col_hbm, row_hbm, vals_hbm, z_hbm, out_hbm,
                  colb, rowb, valsv, rows, acc, isem, gsem, ssem):
        cid = lax.axis_index("c")
        sid = lax.axis_index("s")
        wid = cid * _NS + sid
        zbase = sid * _ROWS_PER_SUB
        pltpu.sync_copy(z_hbm.at[pl.ds(zbase, _ROWS_PER_SUB)],
                        acc.at[pl.ds(zbase, _ROWS_PER_SUB)])
        plsc.subcore_barrier()

        def start_idx(i, b):
            pltpu.async_copy(col_hbm.at[wid, i], colb[b], isem[b])
            pltpu.async_copy(row_hbm.at[wid, i], rowb[b], isem[b])
            pltpu.async_copy(vals_hbm.at[wid, i], valsv[b], isem[b])

        def wait_idx(b):
            pltpu.make_async_copy(col_hbm.at[wid, 0], colb[b], isem[b]).wait()
            pltpu.make_async_copy(row_hbm.at[wid, 0], rowb[b], isem[b]).wait()
            pltpu.make_async_copy(vals_hbm.at[wid, 0], valsv[b], isem[b]).wait()

        def start_gather(b):
            pltpu.async_copy(sup_hbm.at[colb[b]], rows[b], gsem[b])

        def wait_gather(b):
            pltpu.make_async_copy(sup_hbm.at[colb[b]], rows[b], gsem[b]).wait()

        def start_scatter(b):
            pltpu.async_copy(rows[b], acc.at[rowb[b]], ssem[b], add=True)

        def wait_scatter(b):
            pltpu.make_async_copy(rows[b], acc.at[rowb[b]], ssem[b]).wait()

        def scale(b):
            @plsc.parallel_loop(0, _SC_K, unroll=4)
            def _scale(j):
                v = plsc.load_gather(valsv[b], [jnp.full((16,), j, jnp.int32)])
                for t in range(D // 16):
                    rows[b][j, pl.ds(t * 16, 16)] = (
                        rows[b][j, pl.ds(t * 16, 16)] * v)

        start_idx(0, 0)
        start_idx(1, 1)
        wait_idx(0)
        start_gather(0)
        start_idx(2, 2)
        start_idx(3, 3)
        wait_idx(1)
        start_gather(1)

        @pl.loop(0, (_NCH - 1) // _NBUF)
        def _grp(p):
            for b in range(_NBUF):
                i = p * _NBUF + b
                wait_gather(b)
                scale(b)
                start_scatter(b)

                bp = (b + _NBUF - 1) % _NBUF

                @pl.when(i >= 1)
                def _recycle():
                    wait_scatter(bp)

                    @pl.when(i + _NBUF - 1 < _NCH)
                    def _():
                        start_idx(i + _NBUF - 1, bp)

                bg = (b + 2) % _NBUF

                @pl.when(i + 2 < _NCH)
                def _gather_next():
                    wait_idx(bg)
                    start_gather(bg)

        tb = (_NCH - 1) % _NBUF
        wait_gather(tb)
        scale(tb)
        start_scatter(tb)
        wait_scatter((tb + _NBUF - 1) % _NBUF)
        wait_scatter(tb)

        plsc.subcore_barrier()
        pltpu.sync_copy(acc.at[pl.ds(zbase, _ROWS_PER_SUB)],
                        out_hbm.at[cid, pl.ds(zbase, _ROWS_PER_SUB)])

    return sc_kernel(support, col3, row3, vals16, zeros)


def kernel(x, edge_index, adj_values, W, b):
    support = _compute_support(x, W, b.reshape(1, -1).astype(jnp.float32))

    edge3 = edge_index.astype(jnp.int32).reshape(2, _TILES, _NCH, _SC_K)
    vals2 = adj_values.astype(jnp.float32).reshape(_TILES, _NCH, _SC_K)
    zeros = jnp.zeros((_N_PAD, D), jnp.float32)

    partials = _sc_aggregate(support, edge3[1], edge3[0], vals2, zeros)

    return _finalize(partials)


# speedup vs baseline: 11.3766x; 1.0668x over previous
import dataclasses
import functools

import jax
import jax.numpy as jnp
from jax import lax
from jax.experimental import pallas as pl
from jax.experimental.pallas import tpu as pltpu
from jax.experimental.pallas import tpu_sc as plsc

N = 10000
E = 320000
D = 128
MIN_NORM = 1e-15
EPS_F32 = 4e-3

_NC = 2
_NS = 16
_TILES = _NC * _NS
_PER_TILE = E // _TILES
_N_PAD = 10240
_ROWS_PER_SUB = _N_PAD // _NS
_SC_K = 80
_NBUF = 4
_NCH = _PER_TILE // _SC_K


def _rownorm(v):
    return jnp.maximum(jnp.sqrt(jnp.sum(v * v, axis=-1, keepdims=True)), MIN_NORM)


def _proj(v):
    nrm = _rownorm(v)
    maxnorm = 1.0 - EPS_F32
    return jnp.where(nrm > maxnorm, v / nrm * maxnorm, v)


def _support_body(x_ref, w_ref, b_ref, o_ref):
    x = x_ref[...]
    pn = _rownorm(x)
    z = jnp.clip(pn, -1.0 + 1e-7, 1.0 - 1e-7)
    art = 0.5 * jnp.log((1.0 + z) / (1.0 - z))
    u = (art / pn) * x
    s = jnp.dot(u, w_ref[...], preferred_element_type=jnp.float32)
    bias = b_ref[...]
    y = _proj(jnp.tanh(_rownorm(bias)) * bias / _rownorm(bias))
    x2 = jnp.sum(s * s, axis=-1, keepdims=True)
    y2 = jnp.sum(y * y, axis=-1, keepdims=True)
    xy = jnp.sum(s * y, axis=-1, keepdims=True)
    num = (1.0 + 2.0 * xy + y2) * s + (1.0 - x2) * y
    den = jnp.maximum(1.0 + 2.0 * xy + x2 * y2, MIN_NORM)
    o_ref[...] = _proj(num / den)


def _final_body(p_ref, o_ref):
    u = p_ref[0] + p_ref[1]
    un = _rownorm(u)
    e = jnp.tanh(un) * u / un
    o_ref[...] = _proj(e)


_TC_BLK = 1000


def _compute_support(x, W, b2d):
    return pl.pallas_call(
        _support_body,
        grid=(N // _TC_BLK,),
        in_specs=[
            pl.BlockSpec((_TC_BLK, D), lambda i: (i, 0)),
            pl.BlockSpec((D, D), lambda i: (0, 0)),
            pl.BlockSpec((1, D), lambda i: (0, 0)),
        ],
        out_specs=pl.BlockSpec((_TC_BLK, D), lambda i: (i, 0)),
        out_shape=jax.ShapeDtypeStruct((N, D), jnp.float32),
    )(x, W, b2d)


def _finalize(partials):
    return pl.pallas_call(
        _final_body,
        grid=(N // _TC_BLK,),
        in_specs=[pl.BlockSpec((_NC, _TC_BLK, D), lambda i: (0, i, 0))],
        out_specs=pl.BlockSpec((_TC_BLK, D), lambda i: (i, 0)),
        out_shape=jax.ShapeDtypeStruct((N, D), jnp.float32),
    )(partials)


def _sc_aggregate(support, col3, row3, vals16, zeros):
    mesh = plsc.VectorSubcoreMesh(core_axis_name="c", subcore_axis_name="s")
    cp = pltpu.CompilerParams()
    if "needs_layout_passes" in pltpu.CompilerParams.__dataclass_fields__:
        cp = dataclasses.replace(cp, needs_layout_passes=False)

    @functools.partial(
        pl.kernel,
        out_type=jax.ShapeDtypeStruct((_NC, _N_PAD, D), jnp.float32),
        mesh=mesh,
        compiler_params=cp,
        scratch_types=[
            [pltpu.VMEM((_SC_K,), jnp.int32) for _ in range(_NBUF)],
            [pltpu.VMEM((_SC_K,), jnp.int32) for _ in range(_NBUF)],
            [pltpu.VMEM((_SC_K,), jnp.float32) for _ in range(_NBUF)],
            [pltpu.VMEM((_SC_K, D), jnp.float32) for _ in range(_NBUF)],
            pltpu.VMEM_SHARED((_N_PAD, D), jnp.float32),
            [pltpu.SemaphoreType.DMA for _ in range(_NBUF)],
            [pltpu.SemaphoreType.DMA for _ in range(_NBUF)],
            [pltpu.SemaphoreType.DMA for _ in range(_NBUF)],
        ],
    )
    def sc_kernel(sup_hbm, col_hbm, row_hbm, vals_hbm, z_hbm, out_hbm,
                  colb, rowb, valsv, rows, acc, isem, gsem, ssem):
        cid = lax.axis_index("c")
        sid = lax.axis_index("s")
        wid = cid * _NS + sid
        zbase = sid * _ROWS_PER_SUB
        pltpu.sync_copy(z_hbm.at[pl.ds(zbase, _ROWS_PER_SUB)],
                        acc.at[pl.ds(zbase, _ROWS_PER_SUB)])
        plsc.subcore_barrier()

        base = wid * _PER_TILE

        def start_idx(i, b):
            off = base + i * _SC_K
            pltpu.async_copy(col_hbm.at[pl.ds(off, _SC_K)], colb[b], isem[b])
            pltpu.async_copy(row_hbm.at[pl.ds(off, _SC_K)], rowb[b], isem[b])
            pltpu.async_copy(vals_hbm.at[pl.ds(off, _SC_K)], valsv[b], isem[b])

        def wait_idx(b):
            pltpu.make_async_copy(col_hbm.at[pl.ds(0, _SC_K)], colb[b], isem[b]).wait()
            pltpu.make_async_copy(row_hbm.at[pl.ds(0, _SC_K)], rowb[b], isem[b]).wait()
            pltpu.make_async_copy(vals_hbm.at[pl.ds(0, _SC_K)], valsv[b], isem[b]).wait()

        def start_gather(b):
            pltpu.async_copy(sup_hbm.at[colb[b]], rows[b], gsem[b])

        def wait_gather(b):
            pltpu.make_async_copy(sup_hbm.at[colb[b]], rows[b], gsem[b]).wait()

        def start_scatter(b):
            pltpu.async_copy(rows[b], acc.at[rowb[b]], ssem[b], add=True)

        def wait_scatter(b):
            pltpu.make_async_copy(rows[b], acc.at[rowb[b]], ssem[b]).wait()

        def scale(b):
            @plsc.parallel_loop(0, _SC_K, unroll=8)
            def _scale(j):
                v = plsc.load_gather(valsv[b], [jnp.full((16,), j, jnp.int32)])
                for t in range(D // 16):
                    rows[b][j, pl.ds(t * 16, 16)] = (
                        rows[b][j, pl.ds(t * 16, 16)] * v)

        start_idx(0, 0)
        start_idx(1, 1)
        wait_idx(0)
        start_gather(0)
        start_idx(2, 2)
        start_idx(3, 3)
        wait_idx(1)
        start_gather(1)

        @pl.loop(0, (_NCH - 1) // _NBUF)
        def _grp(p):
            for b in range(_NBUF):
                i = p * _NBUF + b
                wait_gather(b)
                scale(b)
                start_scatter(b)

                bp = (b + _NBUF - 1) % _NBUF

                @pl.when(i >= 1)
                def _recycle():
                    wait_scatter(bp)

                    @pl.when(i + _NBUF - 1 < _NCH)
                    def _():
                        start_idx(i + _NBUF - 1, bp)

                bg = (b + 2) % _NBUF

                @pl.when(i + 2 < _NCH)
                def _gather_next():
                    wait_idx(bg)
                    start_gather(bg)

        tb = (_NCH - 1) % _NBUF
        wait_gather(tb)
        scale(tb)
        start_scatter(tb)
        wait_scatter((tb + _NBUF - 1) % _NBUF)
        wait_scatter(tb)

        plsc.subcore_barrier()
        pltpu.sync_copy(acc.at[pl.ds(zbase, _ROWS_PER_SUB)],
                        out_hbm.at[cid, pl.ds(zbase, _ROWS_PER_SUB)])

    return sc_kernel(support, col3, row3, vals16, zeros)


def kernel(x, edge_index, adj_values, W, b):
    support = _compute_support(x, W, b.reshape(1, -1).astype(jnp.float32))

    edge32 = edge_index.astype(jnp.int32)
    zeros = jnp.zeros((_N_PAD, D), jnp.float32)

    partials = _sc_aggregate(support, edge32[1], edge32[0],
                             adj_values.astype(jnp.float32), zeros)

    return _finalize(partials)


# speedup vs baseline: 12.8252x; 1.1273x over previous
import dataclasses
import functools

import jax
import jax.numpy as jnp
from jax import lax
from jax.experimental import pallas as pl
from jax.experimental.pallas import tpu as pltpu
from jax.experimental.pallas import tpu_sc as plsc

N = 10000
E = 320000
D = 128
MIN_NORM = 1e-15
EPS_F32 = 4e-3

_NC = 2
_NS = 16
_TILES = _NC * _NS
_PER_TILE = E // _TILES
_N_PAD = 10240
_ROWS_PER_SUB = _N_PAD // _NS
_SC_K = 80
_NBUF = 4
_NIB = 8
_NCH = _PER_TILE // _SC_K


def _rownorm(v):
    return jnp.maximum(jnp.sqrt(jnp.sum(v * v, axis=-1, keepdims=True)), MIN_NORM)


def _proj(v):
    nrm = _rownorm(v)
    maxnorm = 1.0 - EPS_F32
    return jnp.where(nrm > maxnorm, v / nrm * maxnorm, v)


def _support_body(x_ref, w_ref, b_ref, o_ref):
    x = x_ref[...]
    pn = _rownorm(x)
    z = jnp.clip(pn, -1.0 + 1e-7, 1.0 - 1e-7)
    art = 0.5 * jnp.log((1.0 + z) / (1.0 - z))
    u = (art / pn) * x
    s = jnp.dot(u, w_ref[...], preferred_element_type=jnp.float32)
    bias = b_ref[...]
    y = _proj(jnp.tanh(_rownorm(bias)) * bias / _rownorm(bias))
    x2 = jnp.sum(s * s, axis=-1, keepdims=True)
    y2 = jnp.sum(y * y, axis=-1, keepdims=True)
    xy = jnp.sum(s * y, axis=-1, keepdims=True)
    num = (1.0 + 2.0 * xy + y2) * s + (1.0 - x2) * y
    den = jnp.maximum(1.0 + 2.0 * xy + x2 * y2, MIN_NORM)
    o_ref[...] = _proj(num / den)


def _final_body(p_ref, o_ref):
    u = p_ref[0] + p_ref[1]
    un = _rownorm(u)
    e = jnp.tanh(un) * u / un
    o_ref[...] = _proj(e)


_TC_BLK = 1000


def _compute_support(x, W, b2d):
    return pl.pallas_call(
        _support_body,
        grid=(N // _TC_BLK,),
        in_specs=[
            pl.BlockSpec((_TC_BLK, D), lambda i: (i, 0)),
            pl.BlockSpec((D, D), lambda i: (0, 0)),
            pl.BlockSpec((1, D), lambda i: (0, 0)),
        ],
        out_specs=pl.BlockSpec((_TC_BLK, D), lambda i: (i, 0)),
        out_shape=jax.ShapeDtypeStruct((N, D), jnp.float32),
    )(x, W, b2d)


def _finalize(partials):
    return pl.pallas_call(
        _final_body,
        grid=(N // _TC_BLK,),
        in_specs=[pl.BlockSpec((_NC, _TC_BLK, D), lambda i: (0, i, 0))],
        out_specs=pl.BlockSpec((_TC_BLK, D), lambda i: (i, 0)),
        out_shape=jax.ShapeDtypeStruct((N, D), jnp.float32),
    )(partials)


def _sc_aggregate(support, edges, vals):
    mesh = plsc.VectorSubcoreMesh(core_axis_name="c", subcore_axis_name="s")
    cp = pltpu.CompilerParams()
    if "needs_layout_passes" in pltpu.CompilerParams.__dataclass_fields__:
        cp = dataclasses.replace(cp, needs_layout_passes=False)

    @functools.partial(
        pl.kernel,
        out_type=jax.ShapeDtypeStruct((_NC, _N_PAD, D), jnp.float32),
        mesh=mesh,
        compiler_params=cp,
        scratch_types=[
            [pltpu.VMEM((_SC_K,), jnp.int32) for _ in range(_NIB)],
            [pltpu.VMEM((_SC_K,), jnp.int32) for _ in range(_NIB)],
            [pltpu.VMEM((_SC_K,), jnp.float32) for _ in range(_NIB)],
            [pltpu.VMEM((_SC_K, D), jnp.float32) for _ in range(_NBUF)],
            pltpu.VMEM_SHARED((_N_PAD, D), jnp.float32),
            [pltpu.SemaphoreType.DMA for _ in range(_NIB)],
            [pltpu.SemaphoreType.DMA for _ in range(_NBUF)],
            [pltpu.SemaphoreType.DMA for _ in range(_NBUF)],
        ],
    )
    def sc_kernel(sup_hbm, e_hbm, vals_hbm, out_hbm,
                  colb, rowb, valsv, rows, acc, isem, gsem, ssem):
        cid = lax.axis_index("c")
        sid = lax.axis_index("s")
        wid = cid * _NS + sid
        zbase = sid * _ROWS_PER_SUB
        base = wid * _PER_TILE

        @pl.loop(0, _SC_K)
        def _zrow(j):
            for t in range(D // 16):
                rows[0][j, pl.ds(t * 16, 16)] = jnp.zeros((16,), jnp.float32)

        @pl.loop(0, _ROWS_PER_SUB // _SC_K)
        def _zcopy(q):
            pltpu.sync_copy(rows[0], acc.at[pl.ds(zbase + q * _SC_K, _SC_K)])

        plsc.subcore_barrier()

        def start_idx(i, ib):
            off = base + i * _SC_K
            pltpu.async_copy(e_hbm.at[pl.ds(E + off, _SC_K)], colb[ib], isem[ib])
            pltpu.async_copy(e_hbm.at[pl.ds(off, _SC_K)], rowb[ib], isem[ib])
            pltpu.async_copy(vals_hbm.at[pl.ds(off, _SC_K)], valsv[ib], isem[ib])

        def wait_idx(ib):
            pltpu.make_async_copy(e_hbm.at[pl.ds(0, _SC_K)], colb[ib], isem[ib]).wait()
            pltpu.make_async_copy(e_hbm.at[pl.ds(0, _SC_K)], rowb[ib], isem[ib]).wait()
            pltpu.make_async_copy(vals_hbm.at[pl.ds(0, _SC_K)], valsv[ib], isem[ib]).wait()

        def start_gather(b, ib):
            pltpu.async_copy(sup_hbm.at[colb[ib]], rows[b], gsem[b])

        def wait_gather(b, ib):
            pltpu.make_async_copy(sup_hbm.at[colb[ib]], rows[b], gsem[b]).wait()

        def start_scatter(b, ib):
            pltpu.async_copy(rows[b], acc.at[rowb[ib]], ssem[b], add=True)

        def wait_scatter(b, ib):
            pltpu.make_async_copy(rows[b], acc.at[rowb[ib]], ssem[b]).wait()

        def scale(b, ib):
            @plsc.parallel_loop(0, _SC_K, unroll=8)
            def _scale(j):
                v = plsc.load_gather(valsv[ib], [jnp.full((16,), j, jnp.int32)])
                for t in range(D // 16):
                    rows[b][j, pl.ds(t * 16, 16)] = (
                        rows[b][j, pl.ds(t * 16, 16)] * v)

        def body(i, b, ib, when):
            wait_gather(b, ib)
            scale(b, ib)
            start_scatter(b, ib)
            bp = (b + _NBUF - 1) % _NBUF
            ibp = (ib + _NIB - 1) % _NIB

            def recycle():
                wait_scatter(bp, ibp)

            def gather_next():
                wait_idx((ib + 3) % _NIB)
                start_gather(bp, (ib + 3) % _NIB)

            def idx_next():
                start_idx(i + _NIB - 1, ibp)

            if when:
                if i >= 1:
                    recycle()
                if i + 3 < _NCH:
                    gather_next()
                if i + _NIB - 1 < _NCH:
                    idx_next()
            else:
                pl.when(i >= 1)(recycle)
                pl.when(i + 3 < _NCH)(gather_next)
                pl.when(i + _NIB - 1 < _NCH)(idx_next)

        for c in range(_NIB - 1):
            start_idx(c, c)
        for c in range(3):
            wait_idx(c)
            start_gather(c, c)

        n_main = (_NCH // _NIB) * _NIB

        @pl.loop(0, n_main // _NIB)
        def _grp(p):
            for k in range(_NIB):
                i = p * _NIB + k
                body(i, k % _NBUF, k, when=False)

        for i in range(n_main, _NCH):
            body(i, i % _NBUF, i % _NIB, when=True)

        wait_scatter((_NCH - 1) % _NBUF, (_NCH - 1) % _NIB)

        plsc.subcore_barrier()
        pltpu.sync_copy(acc.at[pl.ds(zbase, _ROWS_PER_SUB)],
                        out_hbm.at[cid, pl.ds(zbase, _ROWS_PER_SUB)])

    return sc_kernel(support, edges, vals)


def kernel(x, edge_index, adj_values, W, b):
    support = _compute_support(x, W, b.reshape(1, -1).astype(jnp.float32))

    edges_flat = edge_index.astype(jnp.int32).reshape(2 * E)
    partials = _sc_aggregate(support, edges_flat, adj_values.astype(jnp.float32))

    return _finalize(partials)
